# baseline (speedup 1.0000x reference)
import jax
import jax.numpy as jnp
from jax import lax
from jax.experimental import pallas as pl
from jax.experimental.pallas import tpu as pltpu
from jax.experimental.pallas import tpu_sc as plsc

_B = 128
_T = 8192
_LANES = 16
_CHUNK = _T // _LANES
_NC = 2
_NS = 16
_NW = _NC * _NS
_RPW = _B // _NW
_OUT_W = _T * 5

_ID_CODE = 0
for _i in range(5):
    _ID_CODE |= (5 * _i) << (5 * _i)


def _compose(prefix, new):
    acc = None
    for i in range(5):
        t = (new >> (5 * i)) & 31
        s = (prefix >> t) & 31
        term = s << (5 * i)
        acc = term if acc is None else acc | term
    return acc


def _sc_body(state_hbm, seq_hbm, ctab_hbm, out_hbm,
             seq_v, out_v, state_v, ctab_v, lane_v, sem0, sem1):
    wid = lax.axis_index("s") * _NC + lax.axis_index("c")
    iota = lax.iota(jnp.int32, _LANES)
    idvec = jnp.full((_LANES,), _ID_CODE, dtype=jnp.int32)
    lane_base = iota * _CHUNK
    obase = iota * (_CHUNK * 5)
    sems = (sem0, sem1)

    pltpu.sync_copy(ctab_hbm, ctab_v)
    lane_v[pl.ds(0, _LANES)] = idvec

    row0 = wid * _RPW
    for j in range(_RPW):
        pltpu.sync_copy(seq_hbm.at[row0 + j], seq_v.at[pl.ds(j * _T, _T)])
    pltpu.sync_copy(state_hbm.at[pl.ds(row0 * 32, _RPW * 32)], state_v)

    def pass1(k, carrys):
        out = []
        for r in range(_RPW):
            idx = lane_base + (r * _T + k)
            u = plsc.load_gather(seq_v, [idx])
            cu = plsc.load_gather(ctab_v, [u])
            c = _compose(carrys[r], cu)
            plsc.store_scatter(seq_v, [idx], c)
            out.append(c)
        return tuple(out)

    tots = lax.fori_loop(0, _CHUNK, pass1, (idvec,) * _RPW, unroll=2)

    for j in range(_RPW):
        x = tots[j]
        for off in (1, 2, 4, 8):
            lane_v[pl.ds(_LANES, _LANES)] = x
            sh = plsc.load_gather(lane_v, [iota + (_LANES - off)])
            x = _compose(sh, x)
        lane_v[pl.ds(_LANES, _LANES)] = x
        lane_off = plsc.load_gather(lane_v, [iota + (_LANES - 1)])

        buf = j % 2
        if j >= 2:
            pltpu.make_async_copy(
                out_v.at[pl.ds(buf * _OUT_W, _OUT_W)],
                out_hbm.at[row0 + j - 2], sems[buf]).wait()
        sbase = j * 32

        def pass2(k, carry):
            idx = lane_base + (j * _T + k)
            local = plsc.load_gather(seq_v, [idx])
            fin = _compose(lane_off, local)
            for i in range(5):
                d5 = (fin >> (5 * i)) & 31
                val = plsc.load_gather(state_v, [d5 + sbase])
                plsc.store_scatter(
                    out_v, [obase + (buf * _OUT_W + (k * 5 + i))], val)
            return carry

        lax.fori_loop(0, _CHUNK, pass2, 0, unroll=2)
        pltpu.make_async_copy(
            out_v.at[pl.ds(buf * _OUT_W, _OUT_W)],
            out_hbm.at[row0 + j], sems[buf]).start()

    for j in (_RPW - 2, _RPW - 1):
        buf = j % 2
        pltpu.make_async_copy(
            out_v.at[pl.ds(buf * _OUT_W, _OUT_W)],
            out_hbm.at[row0 + j], sems[buf]).wait()


def kernel(state, inputs, perm_mats):
    p = jnp.argmax(perm_mats, axis=2).astype(jnp.int32)
    shifts = 5 * jnp.arange(5, dtype=jnp.int32)
    codes = jnp.sum((p * 5) << shifts[None, :], axis=1).astype(jnp.int32)
    ctab = jnp.zeros((128,), jnp.int32).at[:120].set(codes)
    state_pad = jnp.zeros((_B, 32), jnp.float32).at[:, 0:25:5].set(state)
    state_flat = state_pad.reshape(_B * 32)

    mesh = plsc.VectorSubcoreMesh(core_axis_name="c", subcore_axis_name="s")
    fn = pl.kernel(
        _sc_body,
        mesh=mesh,
        compiler_params=pltpu.CompilerParams(needs_layout_passes=False),
        out_type=jax.ShapeDtypeStruct((_B, _OUT_W), jnp.float32),
        scratch_types=[
            pltpu.VMEM((_RPW * _T,), jnp.int32),
            pltpu.VMEM((2 * _OUT_W,), jnp.float32),
            pltpu.VMEM((_RPW * 32,), jnp.float32),
            pltpu.VMEM((128,), jnp.int32),
            pltpu.VMEM((32,), jnp.int32),
            pltpu.SemaphoreType.DMA,
            pltpu.SemaphoreType.DMA,
        ],
    )
    out = fn(state_flat, inputs, ctab)
    return out.reshape(_B, _T, 5)


# speedup vs baseline: 1.4352x; 1.4352x over previous
import jax
import jax.numpy as jnp
from jax import lax
from jax.experimental import pallas as pl
from jax.experimental.pallas import tpu as pltpu
from jax.experimental.pallas import tpu_sc as plsc

_B = 128
_T = 8192
_LANES = 16
_CHUNK = _T // _LANES
_NC = 2
_NS = 16
_NW = _NC * _NS
_RPW = _B // _NW
_OUT_W = _T * 5

_ID_CODE = 0
for _i in range(5):
    _ID_CODE |= (5 * _i) << (5 * _i)


def _compose(prefix, new):
    acc = None
    for i in range(5):
        t = (new >> (5 * i)) & 31
        s = (prefix >> t) & 31
        term = s << (5 * i)
        acc = term if acc is None else acc | term
    return acc


def _sc_body(state_hbm, seq_hbm, ctab_hbm, out_hbm,
             seq_v, out_v, state_v, ctab_v, lane_v, sem0, sem1):
    wid = lax.axis_index("s") * _NC + lax.axis_index("c")
    iota = lax.iota(jnp.int32, _LANES)
    idvec = jnp.full((_LANES,), _ID_CODE, dtype=jnp.int32)
    obase = iota * 5
    sems = (sem0, sem1)

    pltpu.sync_copy(ctab_hbm, ctab_v)
    lane_v[pl.ds(0, _LANES)] = idvec

    row0 = wid * _RPW
    for j in range(_RPW):
        pltpu.sync_copy(seq_hbm.at[row0 + j], seq_v.at[pl.ds(j * _T, _T)])
    pltpu.sync_copy(state_hbm.at[pl.ds(row0 * 32, _RPW * 32)], state_v)

    def pass1(k, carrys):
        out = []
        for r in range(_RPW):
            u = seq_v[pl.ds(r * _T + k * _LANES, _LANES)]
            cu = plsc.load_gather(ctab_v, [u])
            c = _compose(carrys[r], cu)
            seq_v[pl.ds(r * _T + k * _LANES, _LANES)] = c
            out.append(c)
        return tuple(out)

    tots = lax.fori_loop(0, _CHUNK, pass1, (idvec,) * _RPW, unroll=2)

    for j in range(_RPW):
        x = tots[j]
        for off in (1, 2, 4, 8):
            lane_v[pl.ds(_LANES, _LANES)] = x
            sh = plsc.load_gather(lane_v, [iota + (_LANES - off)])
            x = _compose(sh, x)
        lane_v[pl.ds(_LANES, _LANES)] = x
        lane_off = plsc.load_gather(lane_v, [iota + (_LANES - 1)])

        buf = j % 2
        if j >= 2:
            pltpu.make_async_copy(
                out_v.at[pl.ds(buf * _OUT_W, _OUT_W)],
                out_hbm.at[row0 + j - 2], sems[buf]).wait()
        sbase = j * 32

        def pass2(k, carry):
            local = seq_v[pl.ds(j * _T + k * _LANES, _LANES)]
            fin = _compose(lane_off, local)
            for i in range(5):
                d5 = (fin >> (5 * i)) & 31
                val = plsc.load_gather(state_v, [d5 + sbase])
                plsc.store_scatter(
                    out_v, [obase + (buf * _OUT_W + (k * 80 + i))], val)
            return carry

        lax.fori_loop(0, _CHUNK, pass2, 0, unroll=2)
        pltpu.make_async_copy(
            out_v.at[pl.ds(buf * _OUT_W, _OUT_W)],
            out_hbm.at[row0 + j], sems[buf]).start()

    for j in (_RPW - 2, _RPW - 1):
        buf = j % 2
        pltpu.make_async_copy(
            out_v.at[pl.ds(buf * _OUT_W, _OUT_W)],
            out_hbm.at[row0 + j], sems[buf]).wait()


def kernel(state, inputs, perm_mats):
    p = jnp.argmax(perm_mats, axis=2).astype(jnp.int32)
    shifts = 5 * jnp.arange(5, dtype=jnp.int32)
    codes = jnp.sum((p * 5) << shifts[None, :], axis=1).astype(jnp.int32)
    ctab = jnp.zeros((128,), jnp.int32).at[:120].set(codes)
    state_pad = jnp.zeros((_B, 32), jnp.float32).at[:, 0:25:5].set(state)
    state_flat = state_pad.reshape(_B * 32)
    seq = inputs.reshape(_B, _LANES, _CHUNK).swapaxes(1, 2).reshape(_B, _T)

    mesh = plsc.VectorSubcoreMesh(core_axis_name="c", subcore_axis_name="s")
    fn = pl.kernel(
        _sc_body,
        mesh=mesh,
        compiler_params=pltpu.CompilerParams(needs_layout_passes=False),
        out_type=jax.ShapeDtypeStruct((_B, _OUT_W), jnp.float32),
        scratch_types=[
            pltpu.VMEM((_RPW * _T,), jnp.int32),
            pltpu.VMEM((2 * _OUT_W,), jnp.float32),
            pltpu.VMEM((_RPW * 32,), jnp.float32),
            pltpu.VMEM((128,), jnp.int32),
            pltpu.VMEM((32,), jnp.int32),
            pltpu.SemaphoreType.DMA,
            pltpu.SemaphoreType.DMA,
        ],
    )
    out = fn(state_flat, seq, ctab)
    return out.reshape(_B, _CHUNK, _LANES, 5).transpose(0, 2, 1, 3).reshape(
        _B, _T, 5)


# speedup vs baseline: 1.4489x; 1.0096x over previous
import jax
import jax.numpy as jnp
from jax import lax
from jax.experimental import pallas as pl
from jax.experimental.pallas import tpu as pltpu
from jax.experimental.pallas import tpu_sc as plsc

_B = 128
_T = 8192
_LANES = 16
_CHUNK = _T // _LANES
_NC = 2
_NS = 16
_NW = _NC * _NS
_RPW = _B // _NW
_OUT_W = _T * 5

_ID_CODE = 0
for _i in range(5):
    _ID_CODE |= (5 * _i) << (5 * _i)


def _compose(prefix, new):
    acc = None
    for i in range(5):
        t = (new >> (5 * i)) & 31
        s = (prefix >> t) & 31
        term = s << (5 * i)
        acc = term if acc is None else acc | term
    return acc


def _sc_body(state_hbm, seq_hbm, ctab_hbm, out_hbm,
             seq_v, out_v, state_v, ctab_v, lane_v, sem0, sem1):
    wid = lax.axis_index("s") * _NC + lax.axis_index("c")
    iota = lax.iota(jnp.int32, _LANES)
    idvec = jnp.full((_LANES,), _ID_CODE, dtype=jnp.int32)
    obase = iota * 5
    sems = (sem0, sem1)

    pltpu.sync_copy(ctab_hbm, ctab_v)
    lane_v[pl.ds(0, _LANES)] = idvec

    row0 = wid * _RPW
    for j in range(_RPW):
        pltpu.sync_copy(seq_hbm.at[row0 + j], seq_v.at[pl.ds(j * _T, _T)])
    pltpu.sync_copy(state_hbm.at[pl.ds(row0 * 32, _RPW * 32)], state_v)

    def pass1(k, carrys):
        out = []
        for r in range(_RPW):
            u = seq_v[pl.ds(r * _T + k * _LANES, _LANES)]
            cu = plsc.load_gather(ctab_v, [u])
            c = _compose(carrys[r], cu)
            seq_v[pl.ds(r * _T + k * _LANES, _LANES)] = c
            out.append(c)
        return tuple(out)

    tots = lax.fori_loop(0, _CHUNK, pass1, (idvec,) * _RPW, unroll=4)

    for j in range(_RPW):
        x = tots[j]
        for off in (1, 2, 4, 8):
            lane_v[pl.ds(_LANES, _LANES)] = x
            sh = plsc.load_gather(lane_v, [iota + (_LANES - off)])
            x = _compose(sh, x)
        lane_v[pl.ds(_LANES, _LANES)] = x
        lane_off = plsc.load_gather(lane_v, [iota + (_LANES - 1)])

        buf = j % 2
        if j >= 2:
            pltpu.make_async_copy(
                out_v.at[pl.ds(buf * _OUT_W, _OUT_W)],
                out_hbm.at[row0 + j - 2], sems[buf]).wait()
        sbase = j * 32

        def pass2(k, carry):
            local = seq_v[pl.ds(j * _T + k * _LANES, _LANES)]
            fin = _compose(lane_off, local)
            for i in range(5):
                d5 = (fin >> (5 * i)) & 31
                val = plsc.load_gather(state_v, [d5 + sbase])
                plsc.store_scatter(
                    out_v, [obase + (buf * _OUT_W + (k * 80 + i))], val)
            return carry

        lax.fori_loop(0, _CHUNK, pass2, 0, unroll=4)
        pltpu.make_async_copy(
            out_v.at[pl.ds(buf * _OUT_W, _OUT_W)],
            out_hbm.at[row0 + j], sems[buf]).start()

    for j in (_RPW - 2, _RPW - 1):
        buf = j % 2
        pltpu.make_async_copy(
            out_v.at[pl.ds(buf * _OUT_W, _OUT_W)],
            out_hbm.at[row0 + j], sems[buf]).wait()


def kernel(state, inputs, perm_mats):
    p = jnp.argmax(perm_mats, axis=2).astype(jnp.int32)
    shifts = 5 * jnp.arange(5, dtype=jnp.int32)
    codes = jnp.sum((p * 5) << shifts[None, :], axis=1).astype(jnp.int32)
    ctab = jnp.zeros((128,), jnp.int32).at[:120].set(codes)
    state_pad = jnp.zeros((_B, 32), jnp.float32).at[:, 0:25:5].set(state)
    state_flat = state_pad.reshape(_B * 32)
    seq = inputs.reshape(_B, _LANES, _CHUNK).swapaxes(1, 2).reshape(_B, _T)

    mesh = plsc.VectorSubcoreMesh(core_axis_name="c", subcore_axis_name="s")
    fn = pl.kernel(
        _sc_body,
        mesh=mesh,
        compiler_params=pltpu.CompilerParams(needs_layout_passes=False),
        out_type=jax.ShapeDtypeStruct((_B, _OUT_W), jnp.float32),
        scratch_types=[
            pltpu.VMEM((_RPW * _T,), jnp.int32),
            pltpu.VMEM((2 * _OUT_W,), jnp.float32),
            pltpu.VMEM((_RPW * 32,), jnp.float32),
            pltpu.VMEM((128,), jnp.int32),
            pltpu.VMEM((32,), jnp.int32),
            pltpu.SemaphoreType.DMA,
            pltpu.SemaphoreType.DMA,
        ],
    )
    out = fn(state_flat, seq, ctab)
    return out.reshape(_B, _CHUNK, _LANES, 5).transpose(0, 2, 1, 3).reshape(
        _B, _T, 5)
